# baseline (speedup 1.0000x reference)
import functools

import jax
import jax.numpy as jnp
from jax import lax
from jax.experimental import pallas as pl
from jax.experimental.pallas import tpu as pltpu
from jax.experimental.pallas import tpu_sc as plsc

BATCH = 16384
EMBED_DIM = 32
NUM_CORES = 2
NUM_SUBCORES = 16
NUM_WORKERS = NUM_CORES * NUM_SUBCORES
ROWS_PER_WORKER = BATCH // NUM_WORKERS
CHUNK = 128
CHUNKS_PER_WORKER = ROWS_PER_WORKER // CHUNK
L = 16
BLOCK_N = 65536

N_USER = 1000000
N_COURSE = 100000
C_SC = 98304
C_COLS_PER_WORKER = C_SC // NUM_WORKERS
C_TILES_PER_WORKER = C_COLS_PER_WORKER // 128
TAIL_BLOCK = 2048
TAIL_OFF_BLOCKS = C_SC // TAIL_BLOCK


def _proj_tc_kernel(t_ref, w_ref, o_ref):
    o_ref[...] = jnp.sum(t_ref[...] * w_ref[:, 0:1], axis=0, keepdims=True)


def _project_range(table_t, w_col, block_n, off_blocks, n_blocks):
    out = pl.pallas_call(
        _proj_tc_kernel,
        grid=(n_blocks,),
        in_specs=[
            pl.BlockSpec((EMBED_DIM, block_n),
                         lambda g: (0, g + off_blocks)),
            pl.BlockSpec((EMBED_DIM, 128), lambda g: (0, 0)),
        ],
        out_specs=pl.BlockSpec((1, block_n), lambda g: (0, g)),
        out_shape=jax.ShapeDtypeStruct((1, n_blocks * block_n), jnp.float32),
    )(table_t, w_col)
    return out.reshape(n_blocks * block_n)


def _course_sc_kernel(table, wv, p_out, buf, w_v, out_v, sem):
    wid = lax.axis_index("s") * NUM_CORES + lax.axis_index("c")
    base = pl.multiple_of(wid * C_COLS_PER_WORKER, 128)

    pltpu.sync_copy(wv, w_v)
    copies = []
    for r in range(4):
        for t in range(C_TILES_PER_WORKER):
            copies.append(pltpu.async_copy(
                table.at[pl.ds(r * 8, 8), pl.ds(base + t * 128, 128)],
                buf.at[r, t], sem))
    for c in copies:
        c.wait()

    w0 = w_v[0:L]
    w1 = w_v[L:2 * L]

    def body(t, carry):
        for h in range(8):
            acc = buf[0, t, 0, pl.ds(h * L, L)] * w0[0]
            for r in range(4):
                for s in range(8):
                    d = r * 8 + s
                    if d == 0:
                        continue
                    wsc = w0[d] if d < L else w1[d - L]
                    acc = acc + buf[r, t, s, pl.ds(h * L, L)] * wsc
            out_v[pl.ds(t * 128 + h * L, L)] = acc
        return carry

    lax.fori_loop(0, C_TILES_PER_WORKER, body, 0)

    pltpu.sync_copy(out_v, p_out.at[pl.ds(base, C_COLS_PER_WORKER)])


def _gather_sc_kernel(p_u, p_c_lo, p_c_hi, u_idx, c_idx, bias16,
                      out, idx_u, idx_c, idx_lo, idx_hi,
                      g_u, g_lo, g_hi, bias_v, out_v, sem):
    wid = lax.axis_index("s") * NUM_CORES + lax.axis_index("c")
    chunk_base = wid * CHUNKS_PER_WORKER

    pltpu.sync_copy(u_idx.at[pl.ds(chunk_base, CHUNKS_PER_WORKER)], idx_u)
    pltpu.sync_copy(c_idx.at[pl.ds(chunk_base, CHUNKS_PER_WORKER)], idx_c)
    pltpu.sync_copy(bias16, bias_v)

    for k in range(CHUNKS_PER_WORKER):
        for i in range(CHUNK // L):
            s = pl.ds(i * L, L)
            c = idx_c[k, s]
            idx_lo[k, s] = jnp.minimum(c, C_SC - 1)
            idx_hi[k, s] = jnp.maximum(c - C_SC, 0)

    copies = []
    for k in range(CHUNKS_PER_WORKER):
        copies.append(pltpu.async_copy(p_u.at[idx_u.at[k]], g_u.at[k], sem))
        copies.append(pltpu.async_copy(p_c_lo.at[idx_lo.at[k]],
                                       g_lo.at[k], sem))
        copies.append(pltpu.async_copy(p_c_hi.at[idx_hi.at[k]],
                                       g_hi.at[k], sem))
    for c in copies:
        c.wait()

    bias_vec = bias_v[0:L]
    for k in range(CHUNKS_PER_WORKER):
        for i in range(CHUNK // L):
            s = pl.ds(i * L, L)
            cval = jnp.where(idx_c[k, s] < C_SC, g_lo[k, s], g_hi[k, s])
            out_v[pl.ds(k * CHUNK + i * L, L)] = g_u[k, s] + cval + bias_vec

    pltpu.sync_copy(out_v, out.at[pl.ds(wid * ROWS_PER_WORKER,
                                        ROWS_PER_WORKER)])


@jax.jit
def _fm(u_idx, c_idx, table_u_t, table_c_t, w_u, w_c, wv_c, bias16):
    mesh = plsc.VectorSubcoreMesh(core_axis_name="c", subcore_axis_name="s")
    p_c_lo = pl.kernel(
        _course_sc_kernel,
        out_type=jax.ShapeDtypeStruct((C_SC,), jnp.float32),
        mesh=mesh,
        scratch_types=[
            pltpu.VMEM((4, C_TILES_PER_WORKER, 8, 128), jnp.float32),
            pltpu.VMEM((2 * L,), jnp.float32),
            pltpu.VMEM((C_COLS_PER_WORKER,), jnp.float32),
            pltpu.SemaphoreType.DMA,
        ],
        compiler_params=pltpu.CompilerParams(needs_layout_passes=False,
                                             use_tc_tiling_on_sc=True),
    )(table_c_t, wv_c)

    p_u = _project_range(table_u_t, w_u, BLOCK_N, 0,
                         (N_USER + BLOCK_N - 1) // BLOCK_N)
    p_c_hi = _project_range(table_c_t, w_c, TAIL_BLOCK, TAIL_OFF_BLOCKS, 1)

    run = pl.kernel(
        _gather_sc_kernel,
        out_type=jax.ShapeDtypeStruct((BATCH,), jnp.float32),
        mesh=mesh,
        scratch_types=[
            pltpu.VMEM((CHUNKS_PER_WORKER, CHUNK), jnp.int32),
            pltpu.VMEM((CHUNKS_PER_WORKER, CHUNK), jnp.int32),
            pltpu.VMEM((CHUNKS_PER_WORKER, CHUNK), jnp.int32),
            pltpu.VMEM((CHUNKS_PER_WORKER, CHUNK), jnp.int32),
            pltpu.VMEM((CHUNKS_PER_WORKER, CHUNK), jnp.float32),
            pltpu.VMEM((CHUNKS_PER_WORKER, CHUNK), jnp.float32),
            pltpu.VMEM((CHUNKS_PER_WORKER, CHUNK), jnp.float32),
            pltpu.VMEM((L,), jnp.float32),
            pltpu.VMEM((ROWS_PER_WORKER,), jnp.float32),
            pltpu.SemaphoreType.DMA,
        ],
        compiler_params=pltpu.CompilerParams(needs_layout_passes=False,
                                             use_tc_tiling_on_sc=False),
    )
    return run(p_u, p_c_lo, p_c_hi, u_idx, c_idx, bias16)


def kernel(user, course, user_table, course_table, W, b):
    u_idx = user.astype(jnp.int32).reshape(BATCH // CHUNK, CHUNK)
    c_idx = course.astype(jnp.int32).reshape(BATCH // CHUNK, CHUNK)
    w_flat = W.reshape(-1)
    w_u = jnp.broadcast_to(w_flat[:EMBED_DIM, None], (EMBED_DIM, 128))
    w_c = jnp.broadcast_to(w_flat[EMBED_DIM:, None], (EMBED_DIM, 128))
    wv_c = w_flat[EMBED_DIM:]
    bias16 = jnp.broadcast_to(b.reshape(-1), (L,))
    out = _fm(u_idx, c_idx, user_table.T, course_table.T,
              w_u, w_c, wv_c, bias16)
    return out.reshape(BATCH, 1)


# speedup vs baseline: 1.0031x; 1.0031x over previous
import functools

import jax
import jax.numpy as jnp
from jax import lax
from jax.experimental import pallas as pl
from jax.experimental.pallas import tpu as pltpu
from jax.experimental.pallas import tpu_sc as plsc

BATCH = 16384
EMBED_DIM = 32
NUM_CORES = 2
NUM_SUBCORES = 16
NUM_WORKERS = NUM_CORES * NUM_SUBCORES
ROWS_PER_WORKER = BATCH // NUM_WORKERS
CHUNK = 128
CHUNKS_PER_WORKER = ROWS_PER_WORKER // CHUNK
L = 16
BLOCK_N = 65536

N_USER = 1000000
N_COURSE = 100000
C_SC = 98304
C_COLS_PER_WORKER = C_SC // NUM_WORKERS
C_TILES_PER_WORKER = C_COLS_PER_WORKER // 128
TAIL_BLOCK = 2048
TAIL_OFF_BLOCKS = C_SC // TAIL_BLOCK


def _proj_tc_kernel(t_ref, w_ref, o_ref):
    o_ref[...] = jnp.sum(t_ref[...] * w_ref[:, 0:1], axis=0, keepdims=True)


def _project_range(table_t, w_col, block_n, off_blocks, n_blocks):
    out = pl.pallas_call(
        _proj_tc_kernel,
        grid=(n_blocks,),
        in_specs=[
            pl.BlockSpec((EMBED_DIM, block_n),
                         lambda g: (0, g + off_blocks)),
            pl.BlockSpec((EMBED_DIM, 128), lambda g: (0, 0)),
        ],
        out_specs=pl.BlockSpec((1, block_n), lambda g: (0, g)),
        out_shape=jax.ShapeDtypeStruct((1, n_blocks * block_n), jnp.float32),
    )(table_t, w_col)
    return out.reshape(n_blocks * block_n)


def _course_sc_kernel(table, wv, p_out, buf, w_v, out_v, sem):
    wid = lax.axis_index("s") * NUM_CORES + lax.axis_index("c")
    base = pl.multiple_of(wid * C_COLS_PER_WORKER, 128)

    pltpu.sync_copy(wv, w_v)
    copies = []
    for r in range(4):
        copies.append(pltpu.async_copy(
            table.at[pl.ds(r * 8, 8), pl.ds(base, C_COLS_PER_WORKER)],
            buf.at[r], sem))
    for c in copies:
        c.wait()

    w0 = w_v[0:L]
    w1 = w_v[L:2 * L]

    def body(t, carry):
        for h in range(8):
            col = pl.ds(t * 128 + h * L, L)
            acc = buf[0, 0, col] * w0[0]
            for r in range(4):
                for s in range(8):
                    d = r * 8 + s
                    if d == 0:
                        continue
                    wsc = w0[d] if d < L else w1[d - L]
                    acc = acc + buf[r, s, col] * wsc
            out_v[pl.ds(t * 128 + h * L, L)] = acc
        return carry

    lax.fori_loop(0, C_TILES_PER_WORKER, body, 0)

    pltpu.sync_copy(out_v, p_out.at[pl.ds(base, C_COLS_PER_WORKER)])


def _gather_sc_kernel(p_u, p_c_lo, p_c_hi, u_idx, c_idx, bias16,
                      out, idx_u, idx_c, idx_lo, idx_hi,
                      g_u, g_lo, g_hi, bias_v, out_v, sem):
    wid = lax.axis_index("s") * NUM_CORES + lax.axis_index("c")
    chunk_base = wid * CHUNKS_PER_WORKER

    pltpu.sync_copy(u_idx.at[pl.ds(chunk_base, CHUNKS_PER_WORKER)], idx_u)
    pltpu.sync_copy(c_idx.at[pl.ds(chunk_base, CHUNKS_PER_WORKER)], idx_c)
    pltpu.sync_copy(bias16, bias_v)

    for k in range(CHUNKS_PER_WORKER):
        for i in range(CHUNK // L):
            s = pl.ds(i * L, L)
            c = idx_c[k, s]
            idx_lo[k, s] = jnp.minimum(c, C_SC - 1)
            idx_hi[k, s] = jnp.maximum(c - C_SC, 0)

    copies = []
    for k in range(CHUNKS_PER_WORKER):
        copies.append(pltpu.async_copy(p_u.at[idx_u.at[k]], g_u.at[k], sem))
        copies.append(pltpu.async_copy(p_c_lo.at[idx_lo.at[k]],
                                       g_lo.at[k], sem))
        copies.append(pltpu.async_copy(p_c_hi.at[idx_hi.at[k]],
                                       g_hi.at[k], sem))
    for c in copies:
        c.wait()

    bias_vec = bias_v[0:L]
    for k in range(CHUNKS_PER_WORKER):
        for i in range(CHUNK // L):
            s = pl.ds(i * L, L)
            cval = jnp.where(idx_c[k, s] < C_SC, g_lo[k, s], g_hi[k, s])
            out_v[pl.ds(k * CHUNK + i * L, L)] = g_u[k, s] + cval + bias_vec

    pltpu.sync_copy(out_v, out.at[pl.ds(wid * ROWS_PER_WORKER,
                                        ROWS_PER_WORKER)])


@jax.jit
def _fm(u_idx, c_idx, table_u_t, table_c_t, w_u, w_c, wv_c, bias16):
    mesh = plsc.VectorSubcoreMesh(core_axis_name="c", subcore_axis_name="s")
    p_c_lo = pl.kernel(
        _course_sc_kernel,
        out_type=jax.ShapeDtypeStruct((C_SC,), jnp.float32),
        mesh=mesh,
        scratch_types=[
            pltpu.VMEM((4, 8, C_COLS_PER_WORKER), jnp.float32),
            pltpu.VMEM((2 * L,), jnp.float32),
            pltpu.VMEM((C_COLS_PER_WORKER,), jnp.float32),
            pltpu.SemaphoreType.DMA,
        ],
        compiler_params=pltpu.CompilerParams(needs_layout_passes=False,
                                             use_tc_tiling_on_sc=True),
    )(table_c_t, wv_c)

    p_u = _project_range(table_u_t, w_u, BLOCK_N, 0,
                         (N_USER + BLOCK_N - 1) // BLOCK_N)
    p_c_hi = _project_range(table_c_t, w_c, TAIL_BLOCK, TAIL_OFF_BLOCKS, 1)

    run = pl.kernel(
        _gather_sc_kernel,
        out_type=jax.ShapeDtypeStruct((BATCH,), jnp.float32),
        mesh=mesh,
        scratch_types=[
            pltpu.VMEM((CHUNKS_PER_WORKER, CHUNK), jnp.int32),
            pltpu.VMEM((CHUNKS_PER_WORKER, CHUNK), jnp.int32),
            pltpu.VMEM((CHUNKS_PER_WORKER, CHUNK), jnp.int32),
            pltpu.VMEM((CHUNKS_PER_WORKER, CHUNK), jnp.int32),
            pltpu.VMEM((CHUNKS_PER_WORKER, CHUNK), jnp.float32),
            pltpu.VMEM((CHUNKS_PER_WORKER, CHUNK), jnp.float32),
            pltpu.VMEM((CHUNKS_PER_WORKER, CHUNK), jnp.float32),
            pltpu.VMEM((L,), jnp.float32),
            pltpu.VMEM((ROWS_PER_WORKER,), jnp.float32),
            pltpu.SemaphoreType.DMA,
        ],
        compiler_params=pltpu.CompilerParams(needs_layout_passes=False,
                                             use_tc_tiling_on_sc=False),
    )
    return run(p_u, p_c_lo, p_c_hi, u_idx, c_idx, bias16)


def kernel(user, course, user_table, course_table, W, b):
    u_idx = user.astype(jnp.int32).reshape(BATCH // CHUNK, CHUNK)
    c_idx = course.astype(jnp.int32).reshape(BATCH // CHUNK, CHUNK)
    w_flat = W.reshape(-1)
    w_u = jnp.broadcast_to(w_flat[:EMBED_DIM, None], (EMBED_DIM, 128))
    w_c = jnp.broadcast_to(w_flat[EMBED_DIM:, None], (EMBED_DIM, 128))
    wv_c = w_flat[EMBED_DIM:]
    bias16 = jnp.broadcast_to(b.reshape(-1), (L,))
    out = _fm(u_idx, c_idx, user_table.T, course_table.T,
              w_u, w_c, wv_c, bias16)
    return out.reshape(BATCH, 1)


# speedup vs baseline: 1.9307x; 1.9248x over previous
import functools

import jax
import jax.numpy as jnp
from jax import lax
from jax.experimental import pallas as pl
from jax.experimental.pallas import tpu as pltpu
from jax.experimental.pallas import tpu_sc as plsc

BATCH = 16384
EMBED_DIM = 32
NUM_CORES = 2
NUM_SUBCORES = 16
NUM_WORKERS = NUM_CORES * NUM_SUBCORES
ROWS_PER_WORKER = BATCH // NUM_WORKERS
CHUNK = 128
CHUNKS_PER_WORKER = ROWS_PER_WORKER // CHUNK
L = 16
BLOCK_N = 98304


def _proj_tc_kernel(t_ref, w_ref, o_ref):
    o_ref[...] = jnp.sum(t_ref[...] * w_ref[:, 0:1], axis=0, keepdims=True)


def _project(table_t, w_col):
    n = table_t.shape[1]
    grid = (n + BLOCK_N - 1) // BLOCK_N
    out = pl.pallas_call(
        _proj_tc_kernel,
        grid=(grid,),
        in_specs=[
            pl.BlockSpec((EMBED_DIM, BLOCK_N), lambda g: (0, g)),
            pl.BlockSpec((EMBED_DIM, 128), lambda g: (0, 0)),
        ],
        out_specs=pl.BlockSpec((1, BLOCK_N), lambda g: (0, g)),
        out_shape=jax.ShapeDtypeStruct((1, grid * BLOCK_N), jnp.float32),
    )(table_t, w_col)
    return out.reshape(grid * BLOCK_N)


def _gather_sc_kernel(p_u, p_c, u_idx, c_idx, bias16,
                      out, idx_u, idx_c, g_u, g_c, bias_v, out_v, sem):
    wid = lax.axis_index("s") * NUM_CORES + lax.axis_index("c")
    chunk_base = wid * CHUNKS_PER_WORKER

    pltpu.sync_copy(u_idx.at[pl.ds(chunk_base, CHUNKS_PER_WORKER)], idx_u)
    pltpu.sync_copy(c_idx.at[pl.ds(chunk_base, CHUNKS_PER_WORKER)], idx_c)
    pltpu.sync_copy(bias16, bias_v)

    copies = []
    for k in range(CHUNKS_PER_WORKER):
        copies.append(pltpu.async_copy(p_u.at[idx_u.at[k]], g_u.at[k], sem))
        copies.append(pltpu.async_copy(p_c.at[idx_c.at[k]], g_c.at[k], sem))
    for c in copies:
        c.wait()

    bias_vec = bias_v[0:L]
    for k in range(CHUNKS_PER_WORKER):
        for i in range(CHUNK // L):
            s = pl.ds(i * L, L)
            out_v[pl.ds(k * CHUNK + i * L, L)] = g_u[k, s] + g_c[k, s] + bias_vec

    pltpu.sync_copy(out_v, out.at[pl.ds(wid * ROWS_PER_WORKER,
                                        ROWS_PER_WORKER)])


@jax.jit
def _fm(u_idx, c_idx, table_u_t, table_c_t, w_u, w_c, bias16):
    p_u = _project(table_u_t, w_u)
    p_c = _project(table_c_t, w_c)

    mesh = plsc.VectorSubcoreMesh(core_axis_name="c", subcore_axis_name="s")
    run = pl.kernel(
        _gather_sc_kernel,
        out_type=jax.ShapeDtypeStruct((BATCH,), jnp.float32),
        mesh=mesh,
        scratch_types=[
            pltpu.VMEM((CHUNKS_PER_WORKER, CHUNK), jnp.int32),
            pltpu.VMEM((CHUNKS_PER_WORKER, CHUNK), jnp.int32),
            pltpu.VMEM((CHUNKS_PER_WORKER, CHUNK), jnp.float32),
            pltpu.VMEM((CHUNKS_PER_WORKER, CHUNK), jnp.float32),
            pltpu.VMEM((L,), jnp.float32),
            pltpu.VMEM((ROWS_PER_WORKER,), jnp.float32),
            pltpu.SemaphoreType.DMA,
        ],
        compiler_params=pltpu.CompilerParams(needs_layout_passes=False,
                                             use_tc_tiling_on_sc=False),
    )
    return run(p_u, p_c, u_idx, c_idx, bias16)


def kernel(user, course, user_table, course_table, W, b):
    u_idx = user.astype(jnp.int32).reshape(BATCH // CHUNK, CHUNK)
    c_idx = course.astype(jnp.int32).reshape(BATCH // CHUNK, CHUNK)
    w_flat = W.reshape(-1)
    w_u = jnp.broadcast_to(w_flat[:EMBED_DIM, None], (EMBED_DIM, 128))
    w_c = jnp.broadcast_to(w_flat[EMBED_DIM:, None], (EMBED_DIM, 128))
    bias16 = jnp.broadcast_to(b.reshape(-1), (L,))
    out = _fm(u_idx, c_idx, user_table.T, course_table.T, w_u, w_c, bias16)
    return out.reshape(BATCH, 1)


# speedup vs baseline: 2.0115x; 1.0418x over previous
import functools

import jax
import jax.numpy as jnp
from jax import lax
from jax.experimental import pallas as pl
from jax.experimental.pallas import tpu as pltpu
from jax.experimental.pallas import tpu_sc as plsc

BATCH = 16384
EMBED_DIM = 32
NUM_CORES = 2
NUM_SUBCORES = 16
NUM_WORKERS = NUM_CORES * NUM_SUBCORES
ROWS_PER_WORKER = BATCH // NUM_WORKERS
CHUNK = 128
CHUNKS_PER_WORKER = ROWS_PER_WORKER // CHUNK
L = 16
BLOCK_N = 65536


def _proj_tc_kernel(t_ref, w_ref, o_ref):
    o_ref[...] = jnp.sum(t_ref[...] * w_ref[:, 0:1], axis=0, keepdims=True)


def _project(table_t, w_col):
    n = table_t.shape[1]
    grid = (n + BLOCK_N - 1) // BLOCK_N
    out = pl.pallas_call(
        _proj_tc_kernel,
        grid=(grid,),
        in_specs=[
            pl.BlockSpec((EMBED_DIM, BLOCK_N), lambda g: (0, g)),
            pl.BlockSpec((EMBED_DIM, 128), lambda g: (0, 0)),
        ],
        out_specs=pl.BlockSpec((1, BLOCK_N), lambda g: (0, g)),
        out_shape=jax.ShapeDtypeStruct((1, grid * BLOCK_N), jnp.float32),
    )(table_t, w_col)
    return out.reshape(grid * BLOCK_N)


def _gather_sc_kernel(p_u, p_c, u_idx, c_idx, bias16,
                      out, idx_u, idx_c, g_u, g_c, bias_v, out_v, sem):
    wid = lax.axis_index("s") * NUM_CORES + lax.axis_index("c")
    chunk_base = wid * CHUNKS_PER_WORKER

    pltpu.sync_copy(u_idx.at[pl.ds(chunk_base, CHUNKS_PER_WORKER)], idx_u)
    pltpu.sync_copy(c_idx.at[pl.ds(chunk_base, CHUNKS_PER_WORKER)], idx_c)
    pltpu.sync_copy(bias16, bias_v)

    copies = []
    for k in range(CHUNKS_PER_WORKER):
        copies.append(pltpu.async_copy(p_u.at[idx_u.at[k]], g_u.at[k], sem))
        copies.append(pltpu.async_copy(p_c.at[idx_c.at[k]], g_c.at[k], sem))
    for c in copies:
        c.wait()

    bias_vec = bias_v[0:L]
    for k in range(CHUNKS_PER_WORKER):
        for i in range(CHUNK // L):
            s = pl.ds(i * L, L)
            out_v[pl.ds(k * CHUNK + i * L, L)] = g_u[k, s] + g_c[k, s] + bias_vec

    pltpu.sync_copy(out_v, out.at[pl.ds(wid * ROWS_PER_WORKER,
                                        ROWS_PER_WORKER)])


@jax.jit
def _fm(u_idx, c_idx, table_u_t, table_c_t, w_u, w_c, bias16):
    p_u = _project(table_u_t, w_u)
    p_c = _project(table_c_t, w_c)

    mesh = plsc.VectorSubcoreMesh(core_axis_name="c", subcore_axis_name="s")
    run = pl.kernel(
        _gather_sc_kernel,
        out_type=jax.ShapeDtypeStruct((BATCH,), jnp.float32),
        mesh=mesh,
        scratch_types=[
            pltpu.VMEM((CHUNKS_PER_WORKER, CHUNK), jnp.int32),
            pltpu.VMEM((CHUNKS_PER_WORKER, CHUNK), jnp.int32),
            pltpu.VMEM((CHUNKS_PER_WORKER, CHUNK), jnp.float32),
            pltpu.VMEM((CHUNKS_PER_WORKER, CHUNK), jnp.float32),
            pltpu.VMEM((L,), jnp.float32),
            pltpu.VMEM((ROWS_PER_WORKER,), jnp.float32),
            pltpu.SemaphoreType.DMA,
        ],
        compiler_params=pltpu.CompilerParams(needs_layout_passes=False,
                                             use_tc_tiling_on_sc=False),
    )
    return run(p_u, p_c, u_idx, c_idx, bias16)


def kernel(user, course, user_table, course_table, W, b):
    u_idx = user.astype(jnp.int32).reshape(BATCH // CHUNK, CHUNK)
    c_idx = course.astype(jnp.int32).reshape(BATCH // CHUNK, CHUNK)
    w_flat = W.reshape(-1)
    w_u = jnp.broadcast_to(w_flat[:EMBED_DIM, None], (EMBED_DIM, 128))
    w_c = jnp.broadcast_to(w_flat[EMBED_DIM:, None], (EMBED_DIM, 128))
    bias16 = jnp.broadcast_to(b.reshape(-1), (L,))
    out = _fm(u_idx, c_idx, user_table.T, course_table.T, w_u, w_c, bias16)
    return out.reshape(BATCH, 1)


# speedup vs baseline: 2.0280x; 1.0082x over previous
import jax
import jax.numpy as jnp
from jax import lax
from jax.experimental import pallas as pl
from jax.experimental.pallas import tpu as pltpu
from jax.experimental.pallas import tpu_sc as plsc

BATCH = 16384
EMBED_DIM = 32
NUM_CORES = 2
NUM_SUBCORES = 16
NUM_WORKERS = NUM_CORES * NUM_SUBCORES
ROWS_PER_WORKER = BATCH // NUM_WORKERS
CHUNK = 128
CHUNKS_PER_WORKER = ROWS_PER_WORKER // CHUNK
L = 16
BLOCK_N = 73728


def _proj_tc_kernel(t_ref, w_ref, o_ref):
    o_ref[...] = jnp.sum(t_ref[...] * w_ref[:, 0:1], axis=0, keepdims=True)


def _project(table_t, w_col):
    n = table_t.shape[1]
    grid = (n + BLOCK_N - 1) // BLOCK_N
    out = pl.pallas_call(
        _proj_tc_kernel,
        grid=(grid,),
        in_specs=[
            pl.BlockSpec((EMBED_DIM, BLOCK_N), lambda g: (0, g)),
            pl.BlockSpec((EMBED_DIM, 128), lambda g: (0, 0)),
        ],
        out_specs=pl.BlockSpec((1, BLOCK_N), lambda g: (0, g)),
        out_shape=jax.ShapeDtypeStruct((1, grid * BLOCK_N), jnp.float32),
    )(table_t, w_col)
    return out.reshape(grid * BLOCK_N)


def _gather_sc_kernel(p_u, p_c, u_idx, c_idx, bias16,
                      out, idx_u, idx_c, g_u, g_c, bias_v, out_v, sem):
    wid = lax.axis_index("s") * NUM_CORES + lax.axis_index("c")
    chunk_base = wid * CHUNKS_PER_WORKER

    pltpu.sync_copy(u_idx.at[pl.ds(chunk_base, CHUNKS_PER_WORKER)], idx_u)
    pltpu.sync_copy(c_idx.at[pl.ds(chunk_base, CHUNKS_PER_WORKER)], idx_c)
    pltpu.sync_copy(bias16, bias_v)

    copies = []
    for k in range(CHUNKS_PER_WORKER):
        copies.append(pltpu.async_copy(p_u.at[idx_u.at[k]], g_u.at[k], sem))
        copies.append(pltpu.async_copy(p_c.at[idx_c.at[k]], g_c.at[k], sem))
    for c in copies:
        c.wait()

    bias_vec = bias_v[0:L]
    for k in range(CHUNKS_PER_WORKER):
        for i in range(CHUNK // L):
            s = pl.ds(i * L, L)
            out_v[pl.ds(k * CHUNK + i * L, L)] = g_u[k, s] + g_c[k, s] + bias_vec

    pltpu.sync_copy(out_v, out.at[pl.ds(wid * ROWS_PER_WORKER,
                                        ROWS_PER_WORKER)])


@jax.jit
def _fm(u_idx, c_idx, table_u_t, table_c_t, w_u, w_c, bias16):
    p_u = _project(table_u_t, w_u)
    p_c = _project(table_c_t, w_c)

    mesh = plsc.VectorSubcoreMesh(core_axis_name="c", subcore_axis_name="s")
    run = pl.kernel(
        _gather_sc_kernel,
        out_type=jax.ShapeDtypeStruct((BATCH,), jnp.float32),
        mesh=mesh,
        scratch_types=[
            pltpu.VMEM((CHUNKS_PER_WORKER, CHUNK), jnp.int32),
            pltpu.VMEM((CHUNKS_PER_WORKER, CHUNK), jnp.int32),
            pltpu.VMEM((CHUNKS_PER_WORKER, CHUNK), jnp.float32),
            pltpu.VMEM((CHUNKS_PER_WORKER, CHUNK), jnp.float32),
            pltpu.VMEM((L,), jnp.float32),
            pltpu.VMEM((ROWS_PER_WORKER,), jnp.float32),
            pltpu.SemaphoreType.DMA,
        ],
        compiler_params=pltpu.CompilerParams(needs_layout_passes=False,
                                             use_tc_tiling_on_sc=False),
    )
    return run(p_u, p_c, u_idx, c_idx, bias16)


def kernel(user, course, user_table, course_table, W, b):
    u_idx = user.astype(jnp.int32).reshape(BATCH // CHUNK, CHUNK)
    c_idx = course.astype(jnp.int32).reshape(BATCH // CHUNK, CHUNK)
    w_flat = W.reshape(-1)
    w_u = jnp.broadcast_to(w_flat[:EMBED_DIM, None], (EMBED_DIM, 128))
    w_c = jnp.broadcast_to(w_flat[EMBED_DIM:, None], (EMBED_DIM, 128))
    bias16 = jnp.broadcast_to(b.reshape(-1), (L,))
    out = _fm(u_idx, c_idx, user_table.T, course_table.T, w_u, w_c, bias16)
    return out.reshape(BATCH, 1)
